# Initial kernel scaffold; baseline (speedup 1.0000x reference)
import jax
import jax.numpy as jnp
from jax.experimental import pallas as pl


def kernel(features, src_list, dst_list, edge_types, W0, b0, W1, b1, W_ih, W_hh, b_ih, b_hh, W_out, b_out):
    raise NotImplementedError("write your pallas kernel here")


# speedup vs baseline: 34.7482x; 34.7482x over previous
import functools

import jax
import jax.numpy as jnp
from jax import lax
from jax.experimental import pallas as pl
from jax.experimental.pallas import tpu as pltpu
from jax.experimental.pallas import tpu_sc as plsc

_NC = 2
_NS = 16
_NW = _NC * _NS
_L = 16

_CH = 128
_G = 4


def _sc_segment_sum(features, src2d, dst2d, typ2d, n_pad, chunks_per_worker):
    n, d = features.shape
    rps = n_pad // _NS
    groups = chunks_per_worker // _G
    dump = n

    mesh = plsc.VectorSubcoreMesh(
        core_axis_name="c", subcore_axis_name="s",
        num_cores=_NC, num_subcores=_NS)

    @functools.partial(
        pl.kernel,
        out_type=(
            jax.ShapeDtypeStruct((_NC, n_pad, d), jnp.float32),
            jax.ShapeDtypeStruct((_NC, n_pad), jnp.float32),
        ),
        mesh=mesh,
        scratch_types=[
            pltpu.VMEM((_G, _CH), jnp.int32),
            pltpu.VMEM((_G, _CH), jnp.int32),
            pltpu.VMEM((_G, _CH), jnp.int32),
            pltpu.VMEM((_G, _CH), jnp.int32),
            pltpu.VMEM((_G, _CH, 16), jnp.float32),
            pltpu.VMEM((_CH,), jnp.float32),
            pltpu.VMEM_SHARED((n_pad, 16), jnp.float32),
            pltpu.VMEM_SHARED((n_pad,), jnp.float32),
            pltpu.SemaphoreType.DMA,
        ],
        compiler_params=pltpu.CompilerParams(use_tc_tiling_on_sc=False),
    )
    def sc_kernel(feat_hbm, src_hbm, dst_hbm, typ_hbm, zrows_hbm, zdeg_hbm,
                  acc_out, deg_out,
                  srcv, dstv, typv, effv, rowsv, onesv,
                  acc_sh, deg_sh, sem):
        cid = lax.axis_index("c")
        sid = lax.axis_index("s")
        w = cid * _NS + sid
        base = sid * rps

        pltpu.sync_copy(zrows_hbm, acc_sh.at[pl.ds(base, rps)])
        pltpu.sync_copy(zdeg_hbm, deg_sh.at[pl.ds(base, rps)])
        for i in range(_CH // _L):
            onesv[pl.ds(i * _L, _L)] = jnp.full((_L,), 1.0, jnp.float32)
        plsc.subcore_barrier()

        def body(g, carry):
            cbase = w * chunks_per_worker + g * _G
            pltpu.sync_copy(src_hbm.at[pl.ds(cbase, _G)], srcv)
            pltpu.sync_copy(dst_hbm.at[pl.ds(cbase, _G)], dstv)
            pltpu.sync_copy(typ_hbm.at[pl.ds(cbase, _G)], typv)
            for j in range(_G):
                for i in range(_CH // _L):
                    sl = pl.ds(i * _L, _L)
                    t = typv[j, sl]
                    dd = dstv[j, sl]
                    effv[j, sl] = jnp.where(t == 0, dd, dump)
            cps = [pltpu.async_copy(feat_hbm.at[srcv.at[j]], rowsv.at[j], sem)
                   for j in range(_G)]
            for c in cps:
                c.wait()
            for j in range(_G):
                pltpu.sync_copy(rowsv.at[j], acc_sh.at[effv.at[j]], add=True)
                pltpu.sync_copy(onesv, deg_sh.at[effv.at[j]], add=True)
            return carry

        lax.fori_loop(0, groups, body, 0)
        plsc.subcore_barrier()
        pltpu.sync_copy(acc_sh.at[pl.ds(base, rps)],
                        acc_out.at[cid].at[pl.ds(base, rps)])
        pltpu.sync_copy(deg_sh.at[pl.ds(base, rps)],
                        deg_out.at[cid].at[pl.ds(base, rps)])

    zrows = jnp.zeros((rps, 16), jnp.float32)
    zdeg = jnp.zeros((rps,), jnp.float32)
    return sc_kernel(features, src2d, dst2d, typ2d, zrows, zdeg)


def _tc_dense(acc_parts, deg_t, features, misc, fab,
              w0t, b0r, w1t, b1r, wg, bg, woutt, boutr):
    n, d = features.shape
    c = woutt.shape[1]
    r = 2048
    grid = (pl.cdiv(n, r),)

    def body(acc_ref, deg_ref, feat_ref, misc_ref, fab_ref,
             w0_ref, b0_ref, w1_ref, b1_ref, wg_ref, bg_ref,
             wout_ref, bout_ref, out_ref):
        pid = pl.program_id(0)
        acc = acc_ref[0] + acc_ref[1]
        feat = feat_ref[...]
        deg = deg_ref[...]

        count0 = misc_ref[0, 0]
        count1 = misc_ref[0, 1]
        idx_a = misc_ref[0, 2]
        idx_b = misc_ref[0, 3]

        row_a = jnp.dot(fab_ref[0:1, :], w0_ref[...],
                        preferred_element_type=jnp.float32) + b0_ref[...]
        row_b = jnp.dot(fab_ref[1:2, :], w0_ref[...],
                        preferred_element_type=jnp.float32) + b0_ref[...]

        rowf = (lax.broadcasted_iota(jnp.int32, (r, 1), 0)
                + pid * r).astype(jnp.float32)
        corr = ((rowf == idx_a).astype(jnp.float32) * (count0 * row_a)
                + (rowf == idx_b).astype(jnp.float32) * (count1 * row_b))

        reduced = (jnp.dot(acc, w1_ref[...],
                           preferred_element_type=jnp.float32)
                   + deg * b1_ref[...] + corr)

        i_r = jnp.dot(reduced, wg_ref[0], preferred_element_type=jnp.float32) + bg_ref[0:1, :]
        i_z = jnp.dot(reduced, wg_ref[1], preferred_element_type=jnp.float32) + bg_ref[1:2, :]
        i_n = jnp.dot(reduced, wg_ref[2], preferred_element_type=jnp.float32) + bg_ref[2:3, :]
        h_r = jnp.dot(feat, wg_ref[3], preferred_element_type=jnp.float32) + bg_ref[3:4, :]
        h_z = jnp.dot(feat, wg_ref[4], preferred_element_type=jnp.float32) + bg_ref[4:5, :]
        h_n = jnp.dot(feat, wg_ref[5], preferred_element_type=jnp.float32) + bg_ref[5:6, :]

        rr = 1.0 / (1.0 + jnp.exp(-(i_r + h_r)))
        zz = 1.0 / (1.0 + jnp.exp(-(i_z + h_z)))
        nn = jnp.tanh(i_n + rr * h_n)
        h_new = (1.0 - zz) * nn + zz * feat
        out_ref[...] = jnp.dot(h_new, wout_ref[...],
                               preferred_element_type=jnp.float32) + bout_ref[...]

    return pl.pallas_call(
        body,
        grid=grid,
        in_specs=[
            pl.BlockSpec((2, r, d), lambda i: (0, i, 0)),
            pl.BlockSpec((r, 1), lambda i: (i, 0)),
            pl.BlockSpec((r, d), lambda i: (i, 0)),
            pl.BlockSpec((1, 8), lambda i: (0, 0)),
            pl.BlockSpec((2, d), lambda i: (0, 0)),
            pl.BlockSpec((d, d), lambda i: (0, 0)),
            pl.BlockSpec((1, d), lambda i: (0, 0)),
            pl.BlockSpec((d, d), lambda i: (0, 0)),
            pl.BlockSpec((1, d), lambda i: (0, 0)),
            pl.BlockSpec((6, d, d), lambda i: (0, 0, 0)),
            pl.BlockSpec((6, d), lambda i: (0, 0)),
            pl.BlockSpec((d, c), lambda i: (0, 0)),
            pl.BlockSpec((1, c), lambda i: (0, 0)),
        ],
        out_specs=pl.BlockSpec((r, c), lambda i: (i, 0)),
        out_shape=jax.ShapeDtypeStruct((n, c), jnp.float32),
    )(acc_parts, deg_t, features, misc, fab,
      w0t, b0r, w1t, b1r, wg, bg, woutt, boutr)


def kernel(features, src_list, dst_list, edge_types,
           W0, b0, W1, b1, W_ih, W_hh, b_ih, b_hh, W_out, b_out):
    n, d = features.shape
    e = src_list.shape[0]

    unit = _NW * _G * _CH
    e_pad = ((e + unit - 1) // unit) * unit
    pad = e_pad - e
    if pad:
        src_p = jnp.concatenate([src_list, jnp.zeros((pad,), jnp.int32)])
        dst_p = jnp.concatenate([dst_list, jnp.zeros((pad,), jnp.int32)])
        typ_p = jnp.concatenate([edge_types, jnp.ones((pad,), jnp.int32)])
    else:
        src_p, dst_p, typ_p = src_list, dst_list, edge_types
    nchunks = e_pad // _CH
    chunks_per_worker = nchunks // _NW
    src2d = src_p.reshape(nchunks, _CH)
    dst2d = dst_p.reshape(nchunks, _CH)
    typ2d = typ_p.reshape(nchunks, _CH)

    n_pad = ((n + 1 + _NS * 128 - 1) // (_NS * 128)) * (_NS * 128)

    acc_parts, deg_parts = _sc_segment_sum(
        features, src2d, dst2d, typ2d, n_pad, chunks_per_worker)

    deg_t = (deg_parts[0, :n] + deg_parts[1, :n]).reshape(n, 1)
    count1 = deg_parts[0, n] + deg_parts[1, n] - jnp.float32(pad)
    count0 = jnp.float32(e) - count1
    idx_a = dst_list[0].astype(jnp.float32)
    idx_b = dst_list[1].astype(jnp.float32)
    misc = jnp.stack([count0, count1, idx_a, idx_b,
                      jnp.float32(0), jnp.float32(0),
                      jnp.float32(0), jnp.float32(0)]).reshape(1, 8)
    fab = jnp.stack([features[src_list[0]], features[src_list[1]]])

    wg = jnp.stack([W_ih[:d].T, W_ih[d:2 * d].T, W_ih[2 * d:].T,
                    W_hh[:d].T, W_hh[d:2 * d].T, W_hh[2 * d:].T])
    bg = jnp.stack([b_ih[:d], b_ih[d:2 * d], b_ih[2 * d:],
                    b_hh[:d], b_hh[d:2 * d], b_hh[2 * d:]])

    return _tc_dense(acc_parts, deg_t, features, misc, fab,
                     W0.T, b0.reshape(1, d), W1.T, b1.reshape(1, d),
                     wg, bg, W_out.T, b_out.reshape(1, 64))
